# Initial kernel scaffold; baseline (speedup 1.0000x reference)
import functools

import jax
import jax.numpy as jnp
from jax import lax
from jax.experimental import pallas as pl
from jax.experimental.pallas import tpu as pltpu
from jax.experimental.pallas import tpu_sc as plsc

N = 10000
E = 320000
D = 128
G = 64

NC = 2
NS = 16
NW = NC * NS
N_PAD = 10240
ROWS_T = N_PAD // NS
E_W = E // NW
CH = 80
NCH = E_W // CH
BROWS = N_PAD // NW
GP = 80
BLK = 1280
GRID = N_PAD // BLK


def _mesh():
    return plsc.VectorSubcoreMesh(core_axis_name="c", subcore_axis_name="s")


@functools.partial(
    pl.kernel,
    out_type=(jax.ShapeDtypeStruct((NW, N_PAD), jnp.float32),
              jax.ShapeDtypeStruct((NW, GP), jnp.float32)),
    mesh=_mesh(),
    scratch_types=[
        pltpu.VMEM((E_W,), jnp.int32),
        pltpu.VMEM((N_PAD,), jnp.float32),
        pltpu.VMEM((BROWS,), jnp.int32),
        pltpu.VMEM((GP,), jnp.float32),
    ],
)
def _stats(dst_hbm, batch_hbm, zeros1_hbm, deg_out, cnt_out,
           dstv, degv, batchv, cntv):
    c = lax.axis_index("c")
    s = lax.axis_index("s")
    wid = s * NC + c
    pltpu.sync_copy(zeros1_hbm, degv)
    pltpu.sync_copy(zeros1_hbm.at[pl.ds(0, GP)], cntv)
    pltpu.sync_copy(dst_hbm.at[pl.ds(wid * E_W, E_W)], dstv)
    pltpu.sync_copy(batch_hbm.at[pl.ds(wid * BROWS, BROWS)], batchv)
    ones = jnp.ones((16,), jnp.float32)

    def ebody(i, carry):
        idx = dstv[pl.ds(i * 16, 16)]
        plsc.addupdate_scatter(degv, [idx], ones)
        return carry

    lax.fori_loop(0, E_W // 16, ebody, 0)

    def bbody(i, carry):
        idx = batchv[pl.ds(i * 16, 16)]
        plsc.addupdate_scatter(cntv, [idx], ones)
        return carry

    lax.fori_loop(0, BROWS // 16, bbody, 0)
    pltpu.sync_copy(degv, deg_out.at[wid])
    pltpu.sync_copy(cntv, cnt_out.at[wid])


@functools.partial(
    pl.kernel,
    out_type=jax.ShapeDtypeStruct((NC, N_PAD, D), jnp.float32),
    mesh=_mesh(),
    scratch_types=[
        pltpu.VMEM((CH,), jnp.int32),
        pltpu.VMEM((CH,), jnp.int32),
        pltpu.VMEM((CH, D), jnp.float32),
        pltpu.VMEM_SHARED((N_PAD, D), jnp.float32),
        pltpu.SemaphoreType.DMA,
    ],
)
def _msg(y_hbm, src_hbm, dst_hbm, zeros_hbm, out_hbm,
         srcv, dstv, rowsv, acc_sh, sem):
    c = lax.axis_index("c")
    s = lax.axis_index("s")
    wid = s * NC + c
    pltpu.sync_copy(zeros_hbm.at[pl.ds(s * ROWS_T, ROWS_T)],
                    acc_sh.at[pl.ds(s * ROWS_T, ROWS_T)])
    plsc.subcore_barrier()
    base = wid * E_W

    def body(i, carry):
        off = base + i * CH
        pltpu.sync_copy(src_hbm.at[pl.ds(off, CH)], srcv)
        pltpu.sync_copy(dst_hbm.at[pl.ds(off, CH)], dstv)
        pltpu.async_copy(y_hbm.at[srcv], rowsv, sem).wait()
        pltpu.sync_copy(rowsv, acc_sh.at[dstv], add=True)
        return carry

    lax.fori_loop(0, NCH, body, 0)
    plsc.subcore_barrier()
    pltpu.sync_copy(acc_sh.at[pl.ds(s * ROWS_T, ROWS_T)],
                    out_hbm.at[c, pl.ds(s * ROWS_T, ROWS_T)])


@functools.partial(
    pl.kernel,
    out_type=jax.ShapeDtypeStruct((NC, GP, D), jnp.float32),
    mesh=_mesh(),
    scratch_types=[
        pltpu.VMEM((CH,), jnp.int32),
        pltpu.VMEM((CH, D), jnp.float32),
        pltpu.VMEM_SHARED((GP, D), jnp.float32),
    ],
)
def _pool(h_hbm, batch_hbm, zeros_hbm, out_hbm, bv, rowsv, acc_sh):
    c = lax.axis_index("c")
    s = lax.axis_index("s")
    wid = s * NC + c

    @pl.when(s == 0)
    def _zero():
        pltpu.sync_copy(zeros_hbm.at[pl.ds(0, GP)], acc_sh)

    plsc.subcore_barrier()

    def body(i, carry):
        off = wid * BROWS + i * CH
        pltpu.sync_copy(batch_hbm.at[pl.ds(off, CH)], bv)
        pltpu.sync_copy(h_hbm.at[pl.ds(off, CH)], rowsv)
        pltpu.sync_copy(rowsv, acc_sh.at[bv], add=True)
        return carry

    lax.fori_loop(0, BROWS // CH, body, 0)
    plsc.subcore_barrier()

    @pl.when(s == 0)
    def _out():
        pltpu.sync_copy(acc_sh, out_hbm.at[c])


def _prep_body(x_ref, degp_ref, w_ref, y_ref, dinv_ref):
    deg = jnp.sum(degp_ref[...], axis=0) + 1.0
    dinv = lax.rsqrt(jnp.maximum(deg, 1.0))
    xw = jnp.dot(x_ref[...], w_ref[...], preferred_element_type=jnp.float32)
    y_ref[...] = xw * dinv[:, None]
    dinv_ref[...] = dinv[:, None]


_prep = pl.pallas_call(
    _prep_body,
    grid=(GRID,),
    in_specs=[
        pl.BlockSpec((BLK, D), lambda i: (i, 0)),
        pl.BlockSpec((NW, BLK), lambda i: (0, i)),
        pl.BlockSpec((D, D), lambda i: (0, 0)),
    ],
    out_specs=[
        pl.BlockSpec((BLK, D), lambda i: (i, 0)),
        pl.BlockSpec((BLK, 1), lambda i: (i, 0)),
    ],
    out_shape=[
        jax.ShapeDtypeStruct((N_PAD, D), jnp.float32),
        jax.ShapeDtypeStruct((N_PAD, 1), jnp.float32),
    ],
)


def _layer_body(a_ref, y_ref, dinv_ref, b_ref, w_ref, o_ref):
    dinv = dinv_ref[...]
    h = jnp.maximum((a_ref[0] + a_ref[1] + y_ref[...]) * dinv + b_ref[...],
                    0.0)
    o_ref[...] = jnp.dot(h, w_ref[...],
                         preferred_element_type=jnp.float32) * dinv


_layer = pl.pallas_call(
    _layer_body,
    grid=(GRID,),
    in_specs=[
        pl.BlockSpec((NC, BLK, D), lambda i: (0, i, 0)),
        pl.BlockSpec((BLK, D), lambda i: (i, 0)),
        pl.BlockSpec((BLK, 1), lambda i: (i, 0)),
        pl.BlockSpec((1, D), lambda i: (0, 0)),
        pl.BlockSpec((D, D), lambda i: (0, 0)),
    ],
    out_specs=pl.BlockSpec((BLK, D), lambda i: (i, 0)),
    out_shape=jax.ShapeDtypeStruct((N_PAD, D), jnp.float32),
)


def _last_body(a_ref, y_ref, dinv_ref, b_ref, o_ref):
    o_ref[...] = jnp.maximum(
        (a_ref[0] + a_ref[1] + y_ref[...]) * dinv_ref[...] + b_ref[...], 0.0)


_last = pl.pallas_call(
    _last_body,
    grid=(GRID,),
    in_specs=[
        pl.BlockSpec((NC, BLK, D), lambda i: (0, i, 0)),
        pl.BlockSpec((BLK, D), lambda i: (i, 0)),
        pl.BlockSpec((BLK, 1), lambda i: (i, 0)),
        pl.BlockSpec((1, D), lambda i: (0, 0)),
    ],
    out_specs=pl.BlockSpec((BLK, D), lambda i: (i, 0)),
    out_shape=jax.ShapeDtypeStruct((N_PAD, D), jnp.float32),
)


def _head_body(pool_ref, cntp_ref, lw1_ref, lb1_ref, lw2_ref, lb2_ref,
               gamma_ref, beta_ref, ow_ref, ob_ref, out_ref, hid_ref):
    pooled = pool_ref[0, :G, :] + pool_ref[1, :G, :]
    cnt = jnp.sum(cntp_ref[...], axis=0)[:G]
    h = pooled / jnp.maximum(cnt, 1.0)[:, None]
    h = jnp.dot(h, lw1_ref[...], preferred_element_type=jnp.float32) \
        + lb1_ref[...]
    h = jnp.dot(h, lw2_ref[...], preferred_element_type=jnp.float32) \
        + lb2_ref[...]
    mu = jnp.mean(h, axis=0)
    var = jnp.mean((h - mu) ** 2, axis=0)
    h = (h - mu) * lax.rsqrt(var + 1e-5) * gamma_ref[...] + beta_ref[...]
    hidden = jnp.maximum(h, 0.0)
    hid_ref[...] = hidden
    out_ref[...] = jnp.dot(hidden, ow_ref[...],
                           preferred_element_type=jnp.float32) + ob_ref[...]


_head = pl.pallas_call(
    _head_body,
    out_shape=[
        jax.ShapeDtypeStruct((G, 1), jnp.float32),
        jax.ShapeDtypeStruct((G, 64), jnp.float32),
    ],
)


def kernel(x, edge_index, batch_index, W1, b1, W2, b2, W3, b3, W4, b4,
           lw1, lb1, lw2, lb2, gamma, beta, ow, ob):
    src = edge_index[0]
    dst = edge_index[1]
    x_pad = jnp.pad(x, ((0, N_PAD - N), (0, 0)))
    batch_pad = jnp.concatenate(
        [batch_index, jnp.full((N_PAD - N,), G, jnp.int32)])
    zeros2d = jnp.zeros((N_PAD, D), jnp.float32)
    zeros1d = jnp.zeros((N_PAD,), jnp.float32)

    degp, cntp = _stats(dst, batch_pad, zeros1d)
    y, dinv = _prep(x_pad, degp, W1)
    for b, Wn in ((b1, W2), (b2, W3), (b3, W4)):
        acc = _msg(y, src, dst, zeros2d)
        y = _layer(acc, y, dinv, b.reshape(1, D), Wn)
    acc = _msg(y, src, dst, zeros2d)
    h5 = _last(acc, y, dinv, b4.reshape(1, D))
    pool = _pool(h5, batch_pad, zeros2d)
    out, hidden = _head(pool, cntp,
                        lw1, lb1.reshape(1, -1), lw2, lb2.reshape(1, -1),
                        gamma.reshape(1, -1), beta.reshape(1, -1),
                        ow, ob.reshape(1, 1))
    return out, hidden


# speedup vs baseline: 3.0538x; 3.0538x over previous
import functools

import jax
import jax.numpy as jnp
from jax import lax
from jax.experimental import pallas as pl
from jax.experimental.pallas import tpu as pltpu
from jax.experimental.pallas import tpu_sc as plsc

N = 10000
E = 320000
D = 128
G = 64

NC = 2
NS = 16
NW = NC * NS
N_PAD = 10240
ROWS_T = N_PAD // NS
OWN = N_PAD // NW
ACC_ROWS = OWN + 1
ACCW = ACC_ROWS * D
BROWS = N_PAD // NW
CH = 80
NCH = E // NW // CH
GP = 80
BLK = 1280
GRID = N_PAD // BLK
CW = D

BE = 2000
NB = E // BE
NI = BE // 16
FLUSH = 2048
BUF = FLUSH + 256
MC = 128


def _mesh():
    return plsc.VectorSubcoreMesh(core_axis_name="c", subcore_axis_name="s")


@functools.partial(
    pl.kernel,
    out_type=(jax.ShapeDtypeStruct((NC, N_PAD, CW), jnp.float32),
              jax.ShapeDtypeStruct((NC, GP, CW), jnp.float32)),
    mesh=_mesh(),
    scratch_types=[
        pltpu.VMEM((CH,), jnp.int32),
        pltpu.VMEM((CH, CW), jnp.float32),
        pltpu.VMEM_SHARED((N_PAD, CW), jnp.float32),
        pltpu.VMEM_SHARED((GP, CW), jnp.float32),
    ],
)
def _stats(dst_hbm, batch_hbm, ones_hbm, zeros_hbm, deg_out, cnt_out,
           idxv, onesv, deg_sh, cnt_sh):
    c = lax.axis_index("c")
    s = lax.axis_index("s")
    wid = s * NC + c
    pltpu.sync_copy(ones_hbm, onesv)
    pltpu.sync_copy(zeros_hbm.at[pl.ds(s * ROWS_T, ROWS_T)],
                    deg_sh.at[pl.ds(s * ROWS_T, ROWS_T)])

    @pl.when(s == 0)
    def _zero():
        pltpu.sync_copy(zeros_hbm.at[pl.ds(0, GP)], cnt_sh)

    plsc.subcore_barrier()
    base = wid * (E // NW)

    def ebody(i, carry):
        pltpu.sync_copy(dst_hbm.at[pl.ds(base + i * CH, CH)], idxv)
        pltpu.sync_copy(onesv, deg_sh.at[idxv], add=True)
        return carry

    lax.fori_loop(0, NCH, ebody, 0)

    def bbody(i, carry):
        pltpu.sync_copy(batch_hbm.at[pl.ds(wid * BROWS + i * CH, CH)], idxv)
        pltpu.sync_copy(onesv, cnt_sh.at[idxv], add=True)
        return carry

    lax.fori_loop(0, BROWS // CH, bbody, 0)
    plsc.subcore_barrier()
    pltpu.sync_copy(deg_sh.at[pl.ds(s * ROWS_T, ROWS_T)],
                    deg_out.at[c, pl.ds(s * ROWS_T, ROWS_T)])

    @pl.when(s == 0)
    def _out():
        pltpu.sync_copy(cnt_sh, cnt_out.at[c])


@functools.partial(
    pl.kernel,
    out_type=(jax.ShapeDtypeStruct((NW * E,), jnp.int32),
              jax.ShapeDtypeStruct((NW * E,), jnp.int32),
              jax.ShapeDtypeStruct((NW, 16), jnp.int32)),
    mesh=_mesh(),
    scratch_types=[
        pltpu.VMEM((BE,), jnp.int32),
        pltpu.VMEM((BE,), jnp.int32),
        pltpu.VMEM((BUF,), jnp.int32),
        pltpu.VMEM((BUF,), jnp.int32),
        pltpu.VMEM((16,), jnp.int32),
    ],
    compiler_params=pltpu.CompilerParams(needs_layout_passes=False),
)
def _route(src_hbm, dst_hbm, esrc, eoff, ecnt,
           sbuf, dbuf, bs, bo, cbuf):
    c = lax.axis_index("c")
    s = lax.axis_index("s")
    wid = s * NC + c
    lo = wid * OWN
    base = wid * E

    def block(bi, carry):
        pltpu.sync_copy(src_hbm.at[pl.ds(bi * BE, BE)], sbuf)
        pltpu.sync_copy(dst_hbm.at[pl.ds(bi * BE, BE)], dbuf)

        def step(j, carry2):
            cnt, tot = carry2
            d16 = dbuf[pl.ds(j * 16, 16)]
            s16 = sbuf[pl.ds(j * 16, 16)]
            dl = d16 - lo
            m = (dl >= 0) & (dl < OWN)
            plsc.store_compressed(bs.at[pl.ds(cnt, 16)], s16, mask=m)
            plsc.store_compressed(bo.at[pl.ds(cnt, 16)], dl * D, mask=m)
            cnt = cnt + jnp.sum(m.astype(jnp.int32))
            full = cnt >= FLUSH

            @pl.when(full)
            def _flush():
                fo = pl.multiple_of(base + tot, 8)
                pltpu.sync_copy(bs.at[pl.ds(0, FLUSH)],
                                esrc.at[pl.ds(fo, FLUSH)])
                pltpu.sync_copy(bo.at[pl.ds(0, FLUSH)],
                                eoff.at[pl.ds(fo, FLUSH)])
                bs[pl.ds(0, 16)] = bs[pl.ds(FLUSH, 16)]
                bo[pl.ds(0, 16)] = bo[pl.ds(FLUSH, 16)]

            cnt = jnp.where(full, cnt - FLUSH, cnt)
            tot = jnp.where(full, tot + FLUSH, tot)
            return (cnt, tot)

        return lax.fori_loop(0, NI, step, carry)

    cnt, tot = lax.fori_loop(0, NB, block,
                             (jnp.int32(0), jnp.int32(0)))
    zi = jnp.zeros((16,), jnp.int32)
    to = jnp.full((16,), OWN * D, jnp.int32)
    for k in range(MC // 16):
        bs[pl.ds(cnt + k * 16, 16)] = zi
        bo[pl.ds(cnt + k * 16, 16)] = to
    pcnt = jnp.bitwise_and(cnt + (MC - 1), jnp.int32(-MC))

    def fin(k, carry):
        fo = pl.multiple_of(base + tot + k * MC, 8)
        pltpu.sync_copy(bs.at[pl.ds(k * MC, MC)], esrc.at[pl.ds(fo, MC)])
        pltpu.sync_copy(bo.at[pl.ds(k * MC, MC)], eoff.at[pl.ds(fo, MC)])
        return carry

    lax.fori_loop(0, pcnt // MC, fin, 0)
    cbuf[...] = jnp.zeros((16,), jnp.int32) + (tot + pcnt)
    pltpu.sync_copy(cbuf, ecnt.at[wid])


@functools.partial(
    pl.kernel,
    out_type=jax.ShapeDtypeStruct((N_PAD * D,), jnp.float32),
    mesh=_mesh(),
    scratch_types=[
        pltpu.VMEM((ACCW,), jnp.float32),
        pltpu.VMEM((N_PAD + 16,), jnp.float32),
        pltpu.VMEM((MC,), jnp.int32),
        pltpu.VMEM((MC + 16,), jnp.int32),
        pltpu.VMEM((MC + 16,), jnp.int32),
        pltpu.VMEM((MC, D), jnp.float32),
        pltpu.VMEM((16,), jnp.int32),
        pltpu.SemaphoreType.DMA,
    ],
    compiler_params=pltpu.CompilerParams(needs_layout_passes=False),
)
def _msg(xw_hbm, esrc, eoff, ecnt, dinv_hbm, zacc_hbm, out_hbm,
         accv, dinvv, sidx, sibuf, obuf, rows, cbuf, sem):
    c = lax.axis_index("c")
    s = lax.axis_index("s")
    wid = s * NC + c
    base = wid * E
    lo = wid * OWN
    zf = jnp.zeros((16,), jnp.float32)
    pltpu.sync_copy(zacc_hbm, accv)
    pltpu.sync_copy(dinv_hbm, dinvv.at[pl.ds(0, N_PAD)])
    pltpu.sync_copy(ecnt.at[wid], cbuf)
    nch = cbuf[pl.ds(0, 16)][0] // MC

    def chunk(i, carry):
        eb = pl.multiple_of(base + i * MC, 8)
        pltpu.sync_copy(esrc.at[pl.ds(eb, MC)], sidx)
        pltpu.sync_copy(esrc.at[pl.ds(eb, MC)], sibuf.at[pl.ds(0, MC)])
        pltpu.sync_copy(eoff.at[pl.ds(eb, MC)], obuf.at[pl.ds(0, MC)])
        pltpu.async_copy(xw_hbm.at[sidx], rows, sem).wait()

        def edge(j, carry2):
            oj = obuf[pl.ds(j, 16)][0]
            sj = sibuf[pl.ds(j, 16)][0]
            dvs = dinvv[pl.ds(sj, 16)][0]
            dvd = dinvv[pl.ds(lo + (oj >> 7), 16)][0]
            nv = (zf + dvs) * (zf + dvd)
            for k in range(D // 16):
                r = rows[j, pl.ds(k * 16, 16)]
                a = accv[pl.ds(oj + k * 16, 16)]
                accv[pl.ds(oj + k * 16, 16)] = a + r * nv
            return carry2

        lax.fori_loop(0, MC, edge, 0)
        return carry

    lax.fori_loop(0, nch, chunk, 0)
    pltpu.sync_copy(accv.at[pl.ds(0, OWN * D)],
                    out_hbm.at[pl.ds(pl.multiple_of(wid * OWN * D, 8),
                                     OWN * D)])


@functools.partial(
    pl.kernel,
    out_type=jax.ShapeDtypeStruct((NC, GP, D), jnp.float32),
    mesh=_mesh(),
    scratch_types=[
        pltpu.VMEM((CH,), jnp.int32),
        pltpu.VMEM((CH, D), jnp.float32),
        pltpu.VMEM_SHARED((GP, D), jnp.float32),
    ],
)
def _pool(h_hbm, batch_hbm, zeros_hbm, out_hbm, bv, rowsv, acc_sh):
    c = lax.axis_index("c")
    s = lax.axis_index("s")
    wid = s * NC + c

    @pl.when(s == 0)
    def _zero():
        pltpu.sync_copy(zeros_hbm.at[pl.ds(0, GP)], acc_sh)

    plsc.subcore_barrier()

    def body(i, carry):
        off = wid * BROWS + i * CH
        pltpu.sync_copy(batch_hbm.at[pl.ds(off, CH)], bv)
        pltpu.sync_copy(h_hbm.at[pl.ds(off, CH)], rowsv)
        pltpu.sync_copy(rowsv, acc_sh.at[bv], add=True)
        return carry

    lax.fori_loop(0, BROWS // CH, body, 0)
    plsc.subcore_barrier()

    @pl.when(s == 0)
    def _out():
        pltpu.sync_copy(acc_sh, out_hbm.at[c])


def _prep_body(x_ref, degp_ref, w_ref, xw_ref, dinv_ref):
    deg = degp_ref[0, :, 0] + degp_ref[1, :, 0] + 1.0
    dinv = lax.rsqrt(jnp.maximum(deg, 1.0))
    xw_ref[...] = jnp.dot(x_ref[...], w_ref[...],
                          preferred_element_type=jnp.float32)
    dinv_ref[...] = dinv[:, None]


_prep = pl.pallas_call(
    _prep_body,
    grid=(GRID,),
    in_specs=[
        pl.BlockSpec((BLK, D), lambda i: (i, 0)),
        pl.BlockSpec((NC, BLK, CW), lambda i: (0, i, 0)),
        pl.BlockSpec((D, D), lambda i: (0, 0)),
    ],
    out_specs=[
        pl.BlockSpec((BLK, D), lambda i: (i, 0)),
        pl.BlockSpec((BLK, 1), lambda i: (i, 0)),
    ],
    out_shape=[
        jax.ShapeDtypeStruct((N_PAD, D), jnp.float32),
        jax.ShapeDtypeStruct((N_PAD, 1), jnp.float32),
    ],
)


def _node_update(a_ref, xw_ref, dinv_ref, b_ref):
    dinv = dinv_ref[...]
    upd = xw_ref[...] * (dinv * dinv)
    return jnp.maximum((a_ref[...] + upd) + b_ref[...], 0.0)


def _layer_body(a_ref, xw_ref, dinv_ref, b_ref, w_ref, o_ref):
    h = _node_update(a_ref, xw_ref, dinv_ref, b_ref)
    o_ref[...] = jnp.dot(h, w_ref[...], preferred_element_type=jnp.float32)


_layer = pl.pallas_call(
    _layer_body,
    grid=(GRID,),
    in_specs=[
        pl.BlockSpec((BLK, D), lambda i: (i, 0)),
        pl.BlockSpec((BLK, D), lambda i: (i, 0)),
        pl.BlockSpec((BLK, 1), lambda i: (i, 0)),
        pl.BlockSpec((1, D), lambda i: (0, 0)),
        pl.BlockSpec((D, D), lambda i: (0, 0)),
    ],
    out_specs=pl.BlockSpec((BLK, D), lambda i: (i, 0)),
    out_shape=jax.ShapeDtypeStruct((N_PAD, D), jnp.float32),
)


def _last_body(a_ref, xw_ref, dinv_ref, b_ref, o_ref):
    o_ref[...] = _node_update(a_ref, xw_ref, dinv_ref, b_ref)


_last = pl.pallas_call(
    _last_body,
    grid=(GRID,),
    in_specs=[
        pl.BlockSpec((BLK, D), lambda i: (i, 0)),
        pl.BlockSpec((BLK, D), lambda i: (i, 0)),
        pl.BlockSpec((BLK, 1), lambda i: (i, 0)),
        pl.BlockSpec((1, D), lambda i: (0, 0)),
    ],
    out_specs=pl.BlockSpec((BLK, D), lambda i: (i, 0)),
    out_shape=jax.ShapeDtypeStruct((N_PAD, D), jnp.float32),
)


def _head_body(pool_ref, cntp_ref, lw1_ref, lb1_ref, lw2_ref, lb2_ref,
               gamma_ref, beta_ref, ow_ref, ob_ref, out_ref, hid_ref):
    pooled = pool_ref[0, :G, :] + pool_ref[1, :G, :]
    cnt = cntp_ref[0, :G, 0] + cntp_ref[1, :G, 0]
    h = pooled / jnp.maximum(cnt, 1.0)[:, None]
    h = jnp.dot(h, lw1_ref[...], preferred_element_type=jnp.float32) \
        + lb1_ref[...]
    h = jnp.dot(h, lw2_ref[...], preferred_element_type=jnp.float32) \
        + lb2_ref[...]
    mu = jnp.mean(h, axis=0)
    var = jnp.mean((h - mu) ** 2, axis=0)
    h = (h - mu) * lax.rsqrt(var + 1e-5) * gamma_ref[...] + beta_ref[...]
    hidden = jnp.maximum(h, 0.0)
    hid_ref[...] = hidden
    out_ref[...] = jnp.dot(hidden, ow_ref[...],
                           preferred_element_type=jnp.float32) + ob_ref[...]


_head = pl.pallas_call(
    _head_body,
    out_shape=[
        jax.ShapeDtypeStruct((G, 1), jnp.float32),
        jax.ShapeDtypeStruct((G, 64), jnp.float32),
    ],
)


def kernel(x, edge_index, batch_index, W1, b1, W2, b2, W3, b3, W4, b4,
           lw1, lb1, lw2, lb2, gamma, beta, ow, ob):
    src = edge_index[0]
    dst = edge_index[1]
    x_pad = jnp.pad(x, ((0, N_PAD - N), (0, 0)))
    batch_pad = jnp.concatenate(
        [batch_index, jnp.full((N_PAD - N,), G, jnp.int32)])
    zeros2d = jnp.zeros((N_PAD, D), jnp.float32)
    ones2d = jnp.ones((CH, CW), jnp.float32)
    zacc = jnp.zeros((ACCW,), jnp.float32)

    degp, cntp = _stats(dst, batch_pad, ones2d, zeros2d)
    xw, dinv = _prep(x_pad, degp, W1)
    dinv1 = dinv.reshape(N_PAD)
    esrc, eoff, ecnt = _route(src, dst)
    for b, Wn in ((b1, W2), (b2, W3), (b3, W4)):
        acc = _msg(xw, esrc, eoff, ecnt, dinv1, zacc).reshape(N_PAD, D)
        xw = _layer(acc, xw, dinv, b.reshape(1, D), Wn)
    acc = _msg(xw, esrc, eoff, ecnt, dinv1, zacc).reshape(N_PAD, D)
    h5 = _last(acc, xw, dinv, b4.reshape(1, D))
    pool = _pool(h5, batch_pad, zeros2d)
    out, hidden = _head(pool, cntp,
                        lw1, lb1.reshape(1, -1), lw2, lb2.reshape(1, -1),
                        gamma.reshape(1, -1), beta.reshape(1, -1),
                        ow, ob.reshape(1, 1))
    return out, hidden


# speedup vs baseline: 3.1730x; 1.0390x over previous
import functools

import jax
import jax.numpy as jnp
from jax import lax
from jax.experimental import pallas as pl
from jax.experimental.pallas import tpu as pltpu
from jax.experimental.pallas import tpu_sc as plsc

N = 10000
E = 320000
D = 128
G = 64

NC = 2
NS = 16
NW = NC * NS
N_PAD = 10240
ROWS_T = N_PAD // NS
OWN = N_PAD // NW
ACC_ROWS = OWN + 1
ACCW = ACC_ROWS * D
BROWS = N_PAD // NW
CH = 80
NCH = E // NW // CH
GP = 80
BLK = 1280
GRID = N_PAD // BLK
CW = D

BE = 2000
NB = E // BE
NI = BE // 16
FLUSH = 2048
BUF = FLUSH + 256
MC = 128


def _mesh():
    return plsc.VectorSubcoreMesh(core_axis_name="c", subcore_axis_name="s")


@functools.partial(
    pl.kernel,
    out_type=(jax.ShapeDtypeStruct((NC, N_PAD, CW), jnp.float32),
              jax.ShapeDtypeStruct((NC, GP, CW), jnp.float32)),
    mesh=_mesh(),
    scratch_types=[
        pltpu.VMEM((CH,), jnp.int32),
        pltpu.VMEM((CH, CW), jnp.float32),
        pltpu.VMEM_SHARED((N_PAD, CW), jnp.float32),
        pltpu.VMEM_SHARED((GP, CW), jnp.float32),
    ],
)
def _stats(dst_hbm, batch_hbm, ones_hbm, zeros_hbm, deg_out, cnt_out,
           idxv, onesv, deg_sh, cnt_sh):
    c = lax.axis_index("c")
    s = lax.axis_index("s")
    wid = s * NC + c
    pltpu.sync_copy(ones_hbm, onesv)
    pltpu.sync_copy(zeros_hbm.at[pl.ds(s * ROWS_T, ROWS_T)],
                    deg_sh.at[pl.ds(s * ROWS_T, ROWS_T)])

    @pl.when(s == 0)
    def _zero():
        pltpu.sync_copy(zeros_hbm.at[pl.ds(0, GP)], cnt_sh)

    plsc.subcore_barrier()
    base = wid * (E // NW)

    def ebody(i, carry):
        pltpu.sync_copy(dst_hbm.at[pl.ds(base + i * CH, CH)], idxv)
        pltpu.sync_copy(onesv, deg_sh.at[idxv], add=True)
        return carry

    lax.fori_loop(0, NCH, ebody, 0)

    def bbody(i, carry):
        pltpu.sync_copy(batch_hbm.at[pl.ds(wid * BROWS + i * CH, CH)], idxv)
        pltpu.sync_copy(onesv, cnt_sh.at[idxv], add=True)
        return carry

    lax.fori_loop(0, BROWS // CH, bbody, 0)
    plsc.subcore_barrier()
    pltpu.sync_copy(deg_sh.at[pl.ds(s * ROWS_T, ROWS_T)],
                    deg_out.at[c, pl.ds(s * ROWS_T, ROWS_T)])

    @pl.when(s == 0)
    def _out():
        pltpu.sync_copy(cnt_sh, cnt_out.at[c])


@functools.partial(
    pl.kernel,
    out_type=(jax.ShapeDtypeStruct((NW * E,), jnp.int32),
              jax.ShapeDtypeStruct((NW * E,), jnp.int32),
              jax.ShapeDtypeStruct((NW, 16), jnp.int32)),
    mesh=_mesh(),
    scratch_types=[
        pltpu.VMEM((BE,), jnp.int32),
        pltpu.VMEM((BE,), jnp.int32),
        pltpu.VMEM((BUF,), jnp.int32),
        pltpu.VMEM((BUF,), jnp.int32),
        pltpu.VMEM((16,), jnp.int32),
    ],
    compiler_params=pltpu.CompilerParams(needs_layout_passes=False),
)
def _route(src_hbm, dst_hbm, esrc, eoff, ecnt,
           sbuf, dbuf, bs, bo, cbuf):
    c = lax.axis_index("c")
    s = lax.axis_index("s")
    wid = s * NC + c
    lo = wid * OWN
    base = wid * E

    def block(bi, carry):
        pltpu.sync_copy(src_hbm.at[pl.ds(bi * BE, BE)], sbuf)
        pltpu.sync_copy(dst_hbm.at[pl.ds(bi * BE, BE)], dbuf)

        def step(j, carry2):
            cnt, tot = carry2
            d16 = dbuf[pl.ds(j * 16, 16)]
            s16 = sbuf[pl.ds(j * 16, 16)]
            dl = d16 - lo
            m = (dl >= 0) & (dl < OWN)
            plsc.store_compressed(bs.at[pl.ds(cnt, 16)], s16, mask=m)
            plsc.store_compressed(bo.at[pl.ds(cnt, 16)], dl * D, mask=m)
            cnt = cnt + jnp.sum(m.astype(jnp.int32))
            full = cnt >= FLUSH

            @pl.when(full)
            def _flush():
                fo = pl.multiple_of(base + tot, 8)
                pltpu.sync_copy(bs.at[pl.ds(0, FLUSH)],
                                esrc.at[pl.ds(fo, FLUSH)])
                pltpu.sync_copy(bo.at[pl.ds(0, FLUSH)],
                                eoff.at[pl.ds(fo, FLUSH)])
                bs[pl.ds(0, 16)] = bs[pl.ds(FLUSH, 16)]
                bo[pl.ds(0, 16)] = bo[pl.ds(FLUSH, 16)]

            cnt = jnp.where(full, cnt - FLUSH, cnt)
            tot = jnp.where(full, tot + FLUSH, tot)
            return (cnt, tot)

        return lax.fori_loop(0, NI, step, carry)

    cnt, tot = lax.fori_loop(0, NB, block,
                             (jnp.int32(0), jnp.int32(0)))
    zi = jnp.zeros((16,), jnp.int32)
    to = jnp.full((16,), OWN * D, jnp.int32)
    for k in range(2 * MC // 16):
        bs[pl.ds(cnt + k * 16, 16)] = zi
        bo[pl.ds(cnt + k * 16, 16)] = to
    pcnt = jnp.bitwise_and(cnt + (2 * MC - 1), jnp.int32(-(2 * MC)))

    def fin(k, carry):
        fo = pl.multiple_of(base + tot + k * MC, 8)
        pltpu.sync_copy(bs.at[pl.ds(k * MC, MC)], esrc.at[pl.ds(fo, MC)])
        pltpu.sync_copy(bo.at[pl.ds(k * MC, MC)], eoff.at[pl.ds(fo, MC)])
        return carry

    lax.fori_loop(0, pcnt // MC, fin, 0)
    cbuf[...] = jnp.zeros((16,), jnp.int32) + (tot + pcnt)
    pltpu.sync_copy(cbuf, ecnt.at[wid])


@functools.partial(
    pl.kernel,
    out_type=jax.ShapeDtypeStruct((N_PAD * D,), jnp.float32),
    mesh=_mesh(),
    scratch_types=[
        pltpu.VMEM((ACCW,), jnp.float32),
        pltpu.VMEM((N_PAD + 16,), jnp.float32),
        pltpu.VMEM((MC,), jnp.int32),
        pltpu.VMEM((MC,), jnp.int32),
        pltpu.VMEM((MC + 16,), jnp.int32),
        pltpu.VMEM((MC + 16,), jnp.int32),
        pltpu.VMEM((MC + 16,), jnp.int32),
        pltpu.VMEM((MC + 16,), jnp.int32),
        pltpu.VMEM((MC, D), jnp.float32),
        pltpu.VMEM((MC, D), jnp.float32),
        pltpu.VMEM((16,), jnp.int32),
        pltpu.SemaphoreType.DMA,
        pltpu.SemaphoreType.DMA,
    ],
    compiler_params=pltpu.CompilerParams(needs_layout_passes=False),
)
def _msg(xw_hbm, esrc, eoff, ecnt, dinv_hbm, zacc_hbm, out_hbm,
         accv, dinvv, sidx0, sidx1, sibuf0, sibuf1, obuf0, obuf1,
         rows0, rows1, cbuf, sem0, sem1):
    c = lax.axis_index("c")
    s = lax.axis_index("s")
    wid = s * NC + c
    base = wid * E
    lo = wid * OWN
    zf = jnp.zeros((16,), jnp.float32)
    pltpu.sync_copy(zacc_hbm, accv)
    pltpu.sync_copy(dinv_hbm, dinvv.at[pl.ds(0, N_PAD)])
    pltpu.sync_copy(ecnt.at[wid], cbuf)
    npair = cbuf[pl.ds(0, 16)][0] // (2 * MC)

    def fetch(i, sidx, sibuf, obuf, rows, sem):
        eb = pl.multiple_of(base + i * MC, 8)
        pltpu.sync_copy(esrc.at[pl.ds(eb, MC)], sidx)
        pltpu.sync_copy(esrc.at[pl.ds(eb, MC)], sibuf.at[pl.ds(0, MC)])
        pltpu.sync_copy(eoff.at[pl.ds(eb, MC)], obuf.at[pl.ds(0, MC)])
        pltpu.async_copy(xw_hbm.at[sidx], rows, sem)

    def process(sibuf, obuf, rows):
        def edge(j, carry2):
            oj = obuf[pl.ds(j, 16)][0]
            sj = sibuf[pl.ds(j, 16)][0]
            dvs = dinvv[pl.ds(sj, 16)][0]
            dvd = dinvv[pl.ds(lo + (oj >> 7), 16)][0]
            nv = (zf + dvs) * (zf + dvd)
            for k in range(D // 16):
                r = rows[j, pl.ds(k * 16, 16)]
                a = accv[pl.ds(oj + k * 16, 16)]
                accv[pl.ds(oj + k * 16, 16)] = a + r * nv
            return carry2

        lax.fori_loop(0, MC, edge, 0)

    @pl.when(npair > 0)
    def _prologue():
        fetch(0, sidx0, sibuf0, obuf0, rows0, sem0)

    def pair(p, carry):
        fetch(2 * p + 1, sidx1, sibuf1, obuf1, rows1, sem1)
        pltpu.make_async_copy(xw_hbm.at[sidx0], rows0, sem0).wait()
        process(sibuf0, obuf0, rows0)

        @pl.when(p + 1 < npair)
        def _prefetch():
            fetch(2 * p + 2, sidx0, sibuf0, obuf0, rows0, sem0)

        pltpu.make_async_copy(xw_hbm.at[sidx1], rows1, sem1).wait()
        process(sibuf1, obuf1, rows1)
        return carry

    lax.fori_loop(0, npair, pair, 0)
    pltpu.sync_copy(accv.at[pl.ds(0, OWN * D)],
                    out_hbm.at[pl.ds(pl.multiple_of(wid * OWN * D, 8),
                                     OWN * D)])


@functools.partial(
    pl.kernel,
    out_type=jax.ShapeDtypeStruct((NC, GP, D), jnp.float32),
    mesh=_mesh(),
    scratch_types=[
        pltpu.VMEM((CH,), jnp.int32),
        pltpu.VMEM((CH, D), jnp.float32),
        pltpu.VMEM_SHARED((GP, D), jnp.float32),
    ],
)
def _pool(h_hbm, batch_hbm, zeros_hbm, out_hbm, bv, rowsv, acc_sh):
    c = lax.axis_index("c")
    s = lax.axis_index("s")
    wid = s * NC + c

    @pl.when(s == 0)
    def _zero():
        pltpu.sync_copy(zeros_hbm.at[pl.ds(0, GP)], acc_sh)

    plsc.subcore_barrier()

    def body(i, carry):
        off = wid * BROWS + i * CH
        pltpu.sync_copy(batch_hbm.at[pl.ds(off, CH)], bv)
        pltpu.sync_copy(h_hbm.at[pl.ds(off, CH)], rowsv)
        pltpu.sync_copy(rowsv, acc_sh.at[bv], add=True)
        return carry

    lax.fori_loop(0, BROWS // CH, body, 0)
    plsc.subcore_barrier()

    @pl.when(s == 0)
    def _out():
        pltpu.sync_copy(acc_sh, out_hbm.at[c])


def _prep_body(x_ref, degp_ref, w_ref, xw_ref, dinv_ref):
    deg = degp_ref[0, :, 0] + degp_ref[1, :, 0] + 1.0
    dinv = lax.rsqrt(jnp.maximum(deg, 1.0))
    xw_ref[...] = jnp.dot(x_ref[...], w_ref[...],
                          preferred_element_type=jnp.float32)
    dinv_ref[...] = dinv[:, None]


_prep = pl.pallas_call(
    _prep_body,
    grid=(GRID,),
    in_specs=[
        pl.BlockSpec((BLK, D), lambda i: (i, 0)),
        pl.BlockSpec((NC, BLK, CW), lambda i: (0, i, 0)),
        pl.BlockSpec((D, D), lambda i: (0, 0)),
    ],
    out_specs=[
        pl.BlockSpec((BLK, D), lambda i: (i, 0)),
        pl.BlockSpec((BLK, 1), lambda i: (i, 0)),
    ],
    out_shape=[
        jax.ShapeDtypeStruct((N_PAD, D), jnp.float32),
        jax.ShapeDtypeStruct((N_PAD, 1), jnp.float32),
    ],
)


def _node_update(a_ref, xw_ref, dinv_ref, b_ref):
    dinv = dinv_ref[...]
    upd = xw_ref[...] * (dinv * dinv)
    return jnp.maximum((a_ref[...] + upd) + b_ref[...], 0.0)


def _layer_body(a_ref, xw_ref, dinv_ref, b_ref, w_ref, o_ref):
    h = _node_update(a_ref, xw_ref, dinv_ref, b_ref)
    o_ref[...] = jnp.dot(h, w_ref[...], preferred_element_type=jnp.float32)


_layer = pl.pallas_call(
    _layer_body,
    grid=(GRID,),
    in_specs=[
        pl.BlockSpec((BLK, D), lambda i: (i, 0)),
        pl.BlockSpec((BLK, D), lambda i: (i, 0)),
        pl.BlockSpec((BLK, 1), lambda i: (i, 0)),
        pl.BlockSpec((1, D), lambda i: (0, 0)),
        pl.BlockSpec((D, D), lambda i: (0, 0)),
    ],
    out_specs=pl.BlockSpec((BLK, D), lambda i: (i, 0)),
    out_shape=jax.ShapeDtypeStruct((N_PAD, D), jnp.float32),
)


def _last_body(a_ref, xw_ref, dinv_ref, b_ref, o_ref):
    o_ref[...] = _node_update(a_ref, xw_ref, dinv_ref, b_ref)


_last = pl.pallas_call(
    _last_body,
    grid=(GRID,),
    in_specs=[
        pl.BlockSpec((BLK, D), lambda i: (i, 0)),
        pl.BlockSpec((BLK, D), lambda i: (i, 0)),
        pl.BlockSpec((BLK, 1), lambda i: (i, 0)),
        pl.BlockSpec((1, D), lambda i: (0, 0)),
    ],
    out_specs=pl.BlockSpec((BLK, D), lambda i: (i, 0)),
    out_shape=jax.ShapeDtypeStruct((N_PAD, D), jnp.float32),
)


def _head_body(pool_ref, cntp_ref, lw1_ref, lb1_ref, lw2_ref, lb2_ref,
               gamma_ref, beta_ref, ow_ref, ob_ref, out_ref, hid_ref):
    pooled = pool_ref[0, :G, :] + pool_ref[1, :G, :]
    cnt = cntp_ref[0, :G, 0] + cntp_ref[1, :G, 0]
    h = pooled / jnp.maximum(cnt, 1.0)[:, None]
    h = jnp.dot(h, lw1_ref[...], preferred_element_type=jnp.float32) \
        + lb1_ref[...]
    h = jnp.dot(h, lw2_ref[...], preferred_element_type=jnp.float32) \
        + lb2_ref[...]
    mu = jnp.mean(h, axis=0)
    var = jnp.mean((h - mu) ** 2, axis=0)
    h = (h - mu) * lax.rsqrt(var + 1e-5) * gamma_ref[...] + beta_ref[...]
    hidden = jnp.maximum(h, 0.0)
    hid_ref[...] = hidden
    out_ref[...] = jnp.dot(hidden, ow_ref[...],
                           preferred_element_type=jnp.float32) + ob_ref[...]


_head = pl.pallas_call(
    _head_body,
    out_shape=[
        jax.ShapeDtypeStruct((G, 1), jnp.float32),
        jax.ShapeDtypeStruct((G, 64), jnp.float32),
    ],
)


def kernel(x, edge_index, batch_index, W1, b1, W2, b2, W3, b3, W4, b4,
           lw1, lb1, lw2, lb2, gamma, beta, ow, ob):
    src = edge_index[0]
    dst = edge_index[1]
    x_pad = jnp.pad(x, ((0, N_PAD - N), (0, 0)))
    batch_pad = jnp.concatenate(
        [batch_index, jnp.full((N_PAD - N,), G, jnp.int32)])
    zeros2d = jnp.zeros((N_PAD, D), jnp.float32)
    ones2d = jnp.ones((CH, CW), jnp.float32)
    zacc = jnp.zeros((ACCW,), jnp.float32)

    degp, cntp = _stats(dst, batch_pad, ones2d, zeros2d)
    xw, dinv = _prep(x_pad, degp, W1)
    dinv1 = dinv.reshape(N_PAD)
    esrc, eoff, ecnt = _route(src, dst)
    for b, Wn in ((b1, W2), (b2, W3), (b3, W4)):
        acc = _msg(xw, esrc, eoff, ecnt, dinv1, zacc).reshape(N_PAD, D)
        xw = _layer(acc, xw, dinv, b.reshape(1, D), Wn)
    acc = _msg(xw, esrc, eoff, ecnt, dinv1, zacc).reshape(N_PAD, D)
    h5 = _last(acc, xw, dinv, b4.reshape(1, D))
    pool = _pool(h5, batch_pad, zeros2d)
    out, hidden = _head(pool, cntp,
                        lw1, lb1.reshape(1, -1), lw2, lb2.reshape(1, -1),
                        gamma.reshape(1, -1), beta.reshape(1, -1),
                        ow, ob.reshape(1, 1))
    return out, hidden


# speedup vs baseline: 3.4308x; 1.0812x over previous
import functools

import jax
import jax.numpy as jnp
from jax import lax
from jax.experimental import pallas as pl
from jax.experimental.pallas import tpu as pltpu
from jax.experimental.pallas import tpu_sc as plsc

N = 10000
E = 320000
D = 128
G = 64

NC = 2
NS = 16
NW = NC * NS
N_PAD = 10240
ROWS_T = N_PAD // NS
OWN = N_PAD // NW
ACC_ROWS = OWN + 1
ACCW = ACC_ROWS * D
BROWS = N_PAD // NW
CH = 80
NCH = E // NW // CH
GP = 80
BLK = 1280
GRID = N_PAD // BLK
CW = D

BE = 2000
NB = E // BE
NI = BE // 16
FLUSH = 2048
BUF = FLUSH + 256
MC = 128


def _mesh():
    return plsc.VectorSubcoreMesh(core_axis_name="c", subcore_axis_name="s")


@functools.partial(
    pl.kernel,
    out_type=(jax.ShapeDtypeStruct((NC, N_PAD, CW), jnp.float32),
              jax.ShapeDtypeStruct((NC, GP, CW), jnp.float32)),
    mesh=_mesh(),
    scratch_types=[
        pltpu.VMEM((CH,), jnp.int32),
        pltpu.VMEM((CH, CW), jnp.float32),
        pltpu.VMEM_SHARED((N_PAD, CW), jnp.float32),
        pltpu.VMEM_SHARED((GP, CW), jnp.float32),
    ],
)
def _stats(dst_hbm, batch_hbm, ones_hbm, zeros_hbm, deg_out, cnt_out,
           idxv, onesv, deg_sh, cnt_sh):
    c = lax.axis_index("c")
    s = lax.axis_index("s")
    wid = s * NC + c
    pltpu.sync_copy(ones_hbm, onesv)
    pltpu.sync_copy(zeros_hbm.at[pl.ds(s * ROWS_T, ROWS_T)],
                    deg_sh.at[pl.ds(s * ROWS_T, ROWS_T)])

    @pl.when(s == 0)
    def _zero():
        pltpu.sync_copy(zeros_hbm.at[pl.ds(0, GP)], cnt_sh)

    plsc.subcore_barrier()
    base = wid * (E // NW)

    def ebody(i, carry):
        pltpu.sync_copy(dst_hbm.at[pl.ds(base + i * CH, CH)], idxv)
        pltpu.sync_copy(onesv, deg_sh.at[idxv], add=True)
        return carry

    lax.fori_loop(0, NCH, ebody, 0)

    def bbody(i, carry):
        pltpu.sync_copy(batch_hbm.at[pl.ds(wid * BROWS + i * CH, CH)], idxv)
        pltpu.sync_copy(onesv, cnt_sh.at[idxv], add=True)
        return carry

    lax.fori_loop(0, BROWS // CH, bbody, 0)
    plsc.subcore_barrier()
    pltpu.sync_copy(deg_sh.at[pl.ds(s * ROWS_T, ROWS_T)],
                    deg_out.at[c, pl.ds(s * ROWS_T, ROWS_T)])

    @pl.when(s == 0)
    def _out():
        pltpu.sync_copy(cnt_sh, cnt_out.at[c])


@functools.partial(
    pl.kernel,
    out_type=(jax.ShapeDtypeStruct((NW * E,), jnp.int32),
              jax.ShapeDtypeStruct((NW * E,), jnp.int32),
              jax.ShapeDtypeStruct((NW, 16), jnp.int32)),
    mesh=_mesh(),
    scratch_types=[
        pltpu.VMEM((BE,), jnp.int32),
        pltpu.VMEM((BE,), jnp.int32),
        pltpu.VMEM((BUF,), jnp.int32),
        pltpu.VMEM((BUF,), jnp.int32),
        pltpu.VMEM((16,), jnp.int32),
    ],
    compiler_params=pltpu.CompilerParams(needs_layout_passes=False),
)
def _route(src_hbm, dst_hbm, esrc, eoff, ecnt,
           sbuf, dbuf, bs, bo, cbuf):
    c = lax.axis_index("c")
    s = lax.axis_index("s")
    wid = s * NC + c
    lo = wid * OWN
    base = wid * E

    def block(bi, carry):
        pltpu.sync_copy(src_hbm.at[pl.ds(bi * BE, BE)], sbuf)
        pltpu.sync_copy(dst_hbm.at[pl.ds(bi * BE, BE)], dbuf)

        def step(j, carry2):
            cnt, tot = carry2
            d16 = dbuf[pl.ds(j * 16, 16)]
            s16 = sbuf[pl.ds(j * 16, 16)]
            dl = d16 - lo
            m = (dl >= 0) & (dl < OWN)
            plsc.store_compressed(bs.at[pl.ds(cnt, 16)], s16, mask=m)
            plsc.store_compressed(bo.at[pl.ds(cnt, 16)], dl * D, mask=m)
            cnt = cnt + jnp.sum(m.astype(jnp.int32))
            full = cnt >= FLUSH

            @pl.when(full)
            def _flush():
                fo = pl.multiple_of(base + tot, 8)
                pltpu.sync_copy(bs.at[pl.ds(0, FLUSH)],
                                esrc.at[pl.ds(fo, FLUSH)])
                pltpu.sync_copy(bo.at[pl.ds(0, FLUSH)],
                                eoff.at[pl.ds(fo, FLUSH)])
                bs[pl.ds(0, 16)] = bs[pl.ds(FLUSH, 16)]
                bo[pl.ds(0, 16)] = bo[pl.ds(FLUSH, 16)]

            cnt = jnp.where(full, cnt - FLUSH, cnt)
            tot = jnp.where(full, tot + FLUSH, tot)
            return (cnt, tot)

        return lax.fori_loop(0, NI, step, carry)

    cnt, tot = lax.fori_loop(0, NB, block,
                             (jnp.int32(0), jnp.int32(0)))
    zi = jnp.zeros((16,), jnp.int32)
    to = jnp.full((16,), OWN * D, jnp.int32)
    for k in range(2 * MC // 16):
        bs[pl.ds(cnt + k * 16, 16)] = zi
        bo[pl.ds(cnt + k * 16, 16)] = to
    pcnt = jnp.bitwise_and(cnt + (2 * MC - 1), jnp.int32(-(2 * MC)))

    def fin(k, carry):
        fo = pl.multiple_of(base + tot + k * MC, 8)
        pltpu.sync_copy(bs.at[pl.ds(k * MC, MC)], esrc.at[pl.ds(fo, MC)])
        pltpu.sync_copy(bo.at[pl.ds(k * MC, MC)], eoff.at[pl.ds(fo, MC)])
        return carry

    lax.fori_loop(0, pcnt // MC, fin, 0)
    cbuf[...] = jnp.zeros((16,), jnp.int32) + (tot + pcnt)
    pltpu.sync_copy(cbuf, ecnt.at[wid])


@functools.partial(
    pl.kernel,
    out_type=jax.ShapeDtypeStruct((N_PAD * D,), jnp.float32),
    mesh=_mesh(),
    scratch_types=[
        pltpu.VMEM((ACCW,), jnp.float32),
        pltpu.VMEM((N_PAD + 16,), jnp.float32),
        pltpu.VMEM((MC,), jnp.int32),
        pltpu.VMEM((MC,), jnp.int32),
        pltpu.VMEM((MC + 16,), jnp.int32),
        pltpu.VMEM((MC + 16,), jnp.int32),
        pltpu.VMEM((MC + 16,), jnp.int32),
        pltpu.VMEM((MC + 16,), jnp.int32),
        pltpu.VMEM((MC, D), jnp.float32),
        pltpu.VMEM((MC, D), jnp.float32),
        pltpu.VMEM((16,), jnp.int32),
        pltpu.SemaphoreType.DMA,
        pltpu.SemaphoreType.DMA,
    ],
    compiler_params=pltpu.CompilerParams(needs_layout_passes=False),
)
def _msg(xw_hbm, esrc, eoff, ecnt, dinv_hbm, zacc_hbm, out_hbm,
         accv, dinvv, sidx0, sidx1, sibuf0, sibuf1, obuf0, obuf1,
         rows0, rows1, cbuf, sem0, sem1):
    c = lax.axis_index("c")
    s = lax.axis_index("s")
    wid = s * NC + c
    base = wid * E
    lo = wid * OWN
    zf = jnp.zeros((16,), jnp.float32)
    pltpu.sync_copy(zacc_hbm, accv)
    pltpu.sync_copy(dinv_hbm, dinvv.at[pl.ds(0, N_PAD)])
    pltpu.sync_copy(ecnt.at[wid], cbuf)
    npair = cbuf[pl.ds(0, 16)][0] // (2 * MC)

    def fetch(i, sidx, sibuf, obuf, rows, sem):
        eb = pl.multiple_of(base + i * MC, 8)
        pltpu.sync_copy(esrc.at[pl.ds(eb, MC)], sidx)
        pltpu.sync_copy(esrc.at[pl.ds(eb, MC)], sibuf.at[pl.ds(0, MC)])
        pltpu.sync_copy(eoff.at[pl.ds(eb, MC)], obuf.at[pl.ds(0, MC)])
        pltpu.async_copy(xw_hbm.at[sidx], rows, sem)

    def process(sibuf, obuf, rows):
        def one(j):
            oj = obuf[pl.ds(j, 16)][0]
            sj = sibuf[pl.ds(j, 16)][0]
            dvs = dinvv[pl.ds(sj, 16)][0]
            dvd = dinvv[pl.ds(lo + (oj >> 7), 16)][0]
            nv = (zf + dvs) * (zf + dvd)
            return oj, nv

        def edge(j2, carry2):
            oj0, nv0 = one(2 * j2)
            oj1, nv1 = one(2 * j2 + 1)
            for k in range(D // 16):
                r0 = rows[2 * j2, pl.ds(k * 16, 16)]
                a0 = accv[pl.ds(oj0 + k * 16, 16)]
                accv[pl.ds(oj0 + k * 16, 16)] = a0 + r0 * nv0
                r1 = rows[2 * j2 + 1, pl.ds(k * 16, 16)]
                a1 = accv[pl.ds(oj1 + k * 16, 16)]
                accv[pl.ds(oj1 + k * 16, 16)] = a1 + r1 * nv1
            return carry2

        lax.fori_loop(0, MC // 2, edge, 0)

    @pl.when(npair > 0)
    def _prologue():
        fetch(0, sidx0, sibuf0, obuf0, rows0, sem0)

    def pair(p, carry):
        fetch(2 * p + 1, sidx1, sibuf1, obuf1, rows1, sem1)
        pltpu.make_async_copy(xw_hbm.at[sidx0], rows0, sem0).wait()
        process(sibuf0, obuf0, rows0)

        @pl.when(p + 1 < npair)
        def _prefetch():
            fetch(2 * p + 2, sidx0, sibuf0, obuf0, rows0, sem0)

        pltpu.make_async_copy(xw_hbm.at[sidx1], rows1, sem1).wait()
        process(sibuf1, obuf1, rows1)
        return carry

    lax.fori_loop(0, npair, pair, 0)
    pltpu.sync_copy(accv.at[pl.ds(0, OWN * D)],
                    out_hbm.at[pl.ds(pl.multiple_of(wid * OWN * D, 8),
                                     OWN * D)])


@functools.partial(
    pl.kernel,
    out_type=jax.ShapeDtypeStruct((NC, GP, D), jnp.float32),
    mesh=_mesh(),
    scratch_types=[
        pltpu.VMEM((CH,), jnp.int32),
        pltpu.VMEM((CH, D), jnp.float32),
        pltpu.VMEM_SHARED((GP, D), jnp.float32),
    ],
)
def _pool(h_hbm, batch_hbm, zeros_hbm, out_hbm, bv, rowsv, acc_sh):
    c = lax.axis_index("c")
    s = lax.axis_index("s")
    wid = s * NC + c

    @pl.when(s == 0)
    def _zero():
        pltpu.sync_copy(zeros_hbm.at[pl.ds(0, GP)], acc_sh)

    plsc.subcore_barrier()

    def body(i, carry):
        off = wid * BROWS + i * CH
        pltpu.sync_copy(batch_hbm.at[pl.ds(off, CH)], bv)
        pltpu.sync_copy(h_hbm.at[pl.ds(off, CH)], rowsv)
        pltpu.sync_copy(rowsv, acc_sh.at[bv], add=True)
        return carry

    lax.fori_loop(0, BROWS // CH, body, 0)
    plsc.subcore_barrier()

    @pl.when(s == 0)
    def _out():
        pltpu.sync_copy(acc_sh, out_hbm.at[c])


def _prep_body(x_ref, degp_ref, w_ref, xw_ref, dinv_ref):
    deg = degp_ref[0, :, 0] + degp_ref[1, :, 0] + 1.0
    dinv = lax.rsqrt(jnp.maximum(deg, 1.0))
    xw_ref[...] = jnp.dot(x_ref[...], w_ref[...],
                          preferred_element_type=jnp.float32)
    dinv_ref[...] = dinv[:, None]


_prep = pl.pallas_call(
    _prep_body,
    grid=(GRID,),
    in_specs=[
        pl.BlockSpec((BLK, D), lambda i: (i, 0)),
        pl.BlockSpec((NC, BLK, CW), lambda i: (0, i, 0)),
        pl.BlockSpec((D, D), lambda i: (0, 0)),
    ],
    out_specs=[
        pl.BlockSpec((BLK, D), lambda i: (i, 0)),
        pl.BlockSpec((BLK, 1), lambda i: (i, 0)),
    ],
    out_shape=[
        jax.ShapeDtypeStruct((N_PAD, D), jnp.float32),
        jax.ShapeDtypeStruct((N_PAD, 1), jnp.float32),
    ],
)


def _node_update(a_ref, xw_ref, dinv_ref, b_ref):
    dinv = dinv_ref[...]
    upd = xw_ref[...] * (dinv * dinv)
    return jnp.maximum((a_ref[...] + upd) + b_ref[...], 0.0)


def _layer_body(a_ref, xw_ref, dinv_ref, b_ref, w_ref, o_ref):
    h = _node_update(a_ref, xw_ref, dinv_ref, b_ref)
    o_ref[...] = jnp.dot(h, w_ref[...], preferred_element_type=jnp.float32)


_layer = pl.pallas_call(
    _layer_body,
    grid=(GRID,),
    in_specs=[
        pl.BlockSpec((BLK, D), lambda i: (i, 0)),
        pl.BlockSpec((BLK, D), lambda i: (i, 0)),
        pl.BlockSpec((BLK, 1), lambda i: (i, 0)),
        pl.BlockSpec((1, D), lambda i: (0, 0)),
        pl.BlockSpec((D, D), lambda i: (0, 0)),
    ],
    out_specs=pl.BlockSpec((BLK, D), lambda i: (i, 0)),
    out_shape=jax.ShapeDtypeStruct((N_PAD, D), jnp.float32),
)


def _last_body(a_ref, xw_ref, dinv_ref, b_ref, o_ref):
    o_ref[...] = _node_update(a_ref, xw_ref, dinv_ref, b_ref)


_last = pl.pallas_call(
    _last_body,
    grid=(GRID,),
    in_specs=[
        pl.BlockSpec((BLK, D), lambda i: (i, 0)),
        pl.BlockSpec((BLK, D), lambda i: (i, 0)),
        pl.BlockSpec((BLK, 1), lambda i: (i, 0)),
        pl.BlockSpec((1, D), lambda i: (0, 0)),
    ],
    out_specs=pl.BlockSpec((BLK, D), lambda i: (i, 0)),
    out_shape=jax.ShapeDtypeStruct((N_PAD, D), jnp.float32),
)


def _head_body(pool_ref, cntp_ref, lw1_ref, lb1_ref, lw2_ref, lb2_ref,
               gamma_ref, beta_ref, ow_ref, ob_ref, out_ref, hid_ref):
    pooled = pool_ref[0, :G, :] + pool_ref[1, :G, :]
    cnt = cntp_ref[0, :G, 0] + cntp_ref[1, :G, 0]
    h = pooled / jnp.maximum(cnt, 1.0)[:, None]
    h = jnp.dot(h, lw1_ref[...], preferred_element_type=jnp.float32) \
        + lb1_ref[...]
    h = jnp.dot(h, lw2_ref[...], preferred_element_type=jnp.float32) \
        + lb2_ref[...]
    mu = jnp.mean(h, axis=0)
    var = jnp.mean((h - mu) ** 2, axis=0)
    h = (h - mu) * lax.rsqrt(var + 1e-5) * gamma_ref[...] + beta_ref[...]
    hidden = jnp.maximum(h, 0.0)
    hid_ref[...] = hidden
    out_ref[...] = jnp.dot(hidden, ow_ref[...],
                           preferred_element_type=jnp.float32) + ob_ref[...]


_head = pl.pallas_call(
    _head_body,
    out_shape=[
        jax.ShapeDtypeStruct((G, 1), jnp.float32),
        jax.ShapeDtypeStruct((G, 64), jnp.float32),
    ],
)


def kernel(x, edge_index, batch_index, W1, b1, W2, b2, W3, b3, W4, b4,
           lw1, lb1, lw2, lb2, gamma, beta, ow, ob):
    src = edge_index[0]
    dst = edge_index[1]
    x_pad = jnp.pad(x, ((0, N_PAD - N), (0, 0)))
    batch_pad = jnp.concatenate(
        [batch_index, jnp.full((N_PAD - N,), G, jnp.int32)])
    zeros2d = jnp.zeros((N_PAD, D), jnp.float32)
    ones2d = jnp.ones((CH, CW), jnp.float32)
    zacc = jnp.zeros((ACCW,), jnp.float32)

    degp, cntp = _stats(dst, batch_pad, ones2d, zeros2d)
    xw, dinv = _prep(x_pad, degp, W1)
    dinv1 = dinv.reshape(N_PAD)
    esrc, eoff, ecnt = _route(src, dst)
    for b, Wn in ((b1, W2), (b2, W3), (b3, W4)):
        acc = _msg(xw, esrc, eoff, ecnt, dinv1, zacc).reshape(N_PAD, D)
        xw = _layer(acc, xw, dinv, b.reshape(1, D), Wn)
    acc = _msg(xw, esrc, eoff, ecnt, dinv1, zacc).reshape(N_PAD, D)
    h5 = _last(acc, xw, dinv, b4.reshape(1, D))
    pool = _pool(h5, batch_pad, zeros2d)
    out, hidden = _head(pool, cntp,
                        lw1, lb1.reshape(1, -1), lw2, lb2.reshape(1, -1),
                        gamma.reshape(1, -1), beta.reshape(1, -1),
                        ow, ob.reshape(1, 1))
    return out, hidden
